# Initial kernel scaffold; baseline (speedup 1.0000x reference)
import jax
import jax.numpy as jnp
from jax.experimental import pallas as pl


def kernel(x, edge_index, W1, b1, W2, b2, W3, b3):
    raise NotImplementedError("write your pallas kernel here")


# speedup vs baseline: 16.8161x; 16.8161x over previous
import functools

import jax
import jax.numpy as jnp
from jax import lax
from jax.experimental import pallas as pl
from jax.experimental.pallas import tpu as pltpu
from jax.experimental.pallas import tpu_sc as plsc

N = 10000
NPAD = 10240
E = 320000
D_IN = 128
D_H = 128
D_OUT = 5
DOP = 16

NSUB = 16
NW = 32
K = 80
NCH_TOT = E // K
NCH_W = NCH_TOT // NW
RPS = NPAD // NSUB

_MESH = dict(core_axis_name="c", subcore_axis_name="s")


@functools.partial(
    pl.kernel,
    out_type=jax.ShapeDtypeStruct((2, NPAD, 8), jnp.float32),
    mesh=plsc.VectorSubcoreMesh(**_MESH),
    compiler_params=pltpu.CompilerParams(use_tc_tiling_on_sc=False),
    scratch_types=[
        pltpu.VMEM_SHARED((NPAD, 8), jnp.float32),
        pltpu.VMEM((NCH_W, K), jnp.int32),
        pltpu.VMEM((K, 8), jnp.float32),
    ],
)
def _deg_kernel(dstc_hbm, ones_hbm, zeros_hbm, out_hbm, acc_s, didx_v, ones_v):
    cid = lax.axis_index("c")
    sid = lax.axis_index("s")
    wid = cid * NSUB + sid
    pltpu.sync_copy(zeros_hbm, acc_s.at[pl.ds(sid * RPS, RPS)])
    pltpu.sync_copy(ones_hbm, ones_v)
    pltpu.sync_copy(dstc_hbm.at[wid], didx_v)
    plsc.subcore_barrier()

    def body(j, carry):
        pltpu.sync_copy(ones_v, acc_s.at[didx_v.at[j]], add=True)
        return carry

    lax.fori_loop(0, NCH_W, body, 0)
    plsc.subcore_barrier()
    pltpu.sync_copy(acc_s.at[pl.ds(sid * RPS, RPS)],
                    out_hbm.at[cid, pl.ds(sid * RPS, RPS)])


def _make_agg(d):
    @functools.partial(
        pl.kernel,
        out_type=jax.ShapeDtypeStruct((2, NPAD, d), jnp.float32),
        mesh=plsc.VectorSubcoreMesh(**_MESH),
        compiler_params=pltpu.CompilerParams(use_tc_tiling_on_sc=(d % 128 == 0)),
        scratch_types=[
            pltpu.VMEM_SHARED((NPAD, d), jnp.float32),
            pltpu.VMEM((NCH_W, K), jnp.int32),
            pltpu.VMEM((NCH_W, K), jnp.int32),
            pltpu.VMEM((K, d), jnp.float32),
            pltpu.SemaphoreType.DMA,
        ],
    )
    def agg(hs_hbm, srcc_hbm, dstc_hbm, zeros_hbm, out_hbm,
            acc_s, sidx_v, didx_v, rows_v, sem):
        cid = lax.axis_index("c")
        sid = lax.axis_index("s")
        wid = cid * NSUB + sid
        pltpu.sync_copy(zeros_hbm, acc_s.at[pl.ds(sid * RPS, RPS)])
        pltpu.sync_copy(srcc_hbm.at[wid], sidx_v)
        pltpu.sync_copy(dstc_hbm.at[wid], didx_v)
        plsc.subcore_barrier()

        def body(j, carry):
            pltpu.async_copy(hs_hbm.at[sidx_v.at[j]], rows_v, sem).wait()
            pltpu.sync_copy(rows_v, acc_s.at[didx_v.at[j]], add=True)
            return carry

        lax.fori_loop(0, NCH_W, body, 0)
        plsc.subcore_barrier()
        pltpu.sync_copy(acc_s.at[pl.ds(sid * RPS, RPS)],
                        out_hbm.at[cid, pl.ds(sid * RPS, RPS)])

    return agg


_agg128 = _make_agg(D_H)
_agg16 = _make_agg(DOP)


BM = 256
GRID = NPAD // BM


def _tc1_body(x_ref, deg_ref, w_ref, hs_ref, dinv_ref):
    deg = deg_ref[0] + deg_ref[1] + 1.0
    dv = lax.rsqrt(deg)
    h = jnp.dot(x_ref[...], w_ref[...], preferred_element_type=jnp.float32)
    hs_ref[...] = h * dv[:, :1]
    dinv_ref[...] = dv


def _tc_mid_body(parts_ref, hsp_ref, dinv_ref, b_ref, w_ref, out_ref):
    dv = dinv_ref[...][:, :1]
    a = (parts_ref[0] + parts_ref[1] + hsp_ref[...]) * dv + b_ref[...]
    a = jnp.maximum(a, 0.0)
    out_ref[...] = jnp.dot(a, w_ref[...], preferred_element_type=jnp.float32) * dv


def _tc_fin_body(parts_ref, hsp_ref, dinv_ref, b_ref, out_ref):
    dv = dinv_ref[...][:, :1]
    out_ref[...] = (parts_ref[0] + parts_ref[1] + hsp_ref[...]) * dv + b_ref[...]


def _tc1(xp, degp, W1):
    return pl.pallas_call(
        _tc1_body,
        grid=(GRID,),
        in_specs=[
            pl.BlockSpec((BM, D_IN), lambda i: (i, 0)),
            pl.BlockSpec((2, BM, 8), lambda i: (0, i, 0)),
            pl.BlockSpec((D_IN, D_H), lambda i: (0, 0)),
        ],
        out_specs=[
            pl.BlockSpec((BM, D_H), lambda i: (i, 0)),
            pl.BlockSpec((BM, 8), lambda i: (i, 0)),
        ],
        out_shape=[
            jax.ShapeDtypeStruct((NPAD, D_H), jnp.float32),
            jax.ShapeDtypeStruct((NPAD, 8), jnp.float32),
        ],
    )(xp, degp, W1)


def _tc_mid(parts, hsp, dinv, b, W, d_out):
    d_in = hsp.shape[1]
    return pl.pallas_call(
        _tc_mid_body,
        grid=(GRID,),
        in_specs=[
            pl.BlockSpec((2, BM, d_in), lambda i: (0, i, 0)),
            pl.BlockSpec((BM, d_in), lambda i: (i, 0)),
            pl.BlockSpec((BM, 8), lambda i: (i, 0)),
            pl.BlockSpec((1, d_in), lambda i: (0, 0)),
            pl.BlockSpec((d_in, d_out), lambda i: (0, 0)),
        ],
        out_specs=pl.BlockSpec((BM, d_out), lambda i: (i, 0)),
        out_shape=jax.ShapeDtypeStruct((NPAD, d_out), jnp.float32),
    )(parts, hsp, dinv, b, W)


def _tc_fin(parts, hsp, dinv, b):
    return pl.pallas_call(
        _tc_fin_body,
        grid=(GRID,),
        in_specs=[
            pl.BlockSpec((2, BM, DOP), lambda i: (0, i, 0)),
            pl.BlockSpec((BM, DOP), lambda i: (i, 0)),
            pl.BlockSpec((BM, 8), lambda i: (i, 0)),
            pl.BlockSpec((1, DOP), lambda i: (0, 0)),
        ],
        out_specs=pl.BlockSpec((BM, DOP), lambda i: (i, 0)),
        out_shape=jax.ShapeDtypeStruct((NPAD, DOP), jnp.float32),
    )(parts, hsp, dinv, b)


@jax.jit
def kernel(x, edge_index, W1, b1, W2, b2, W3, b3):
    srcc = edge_index[0].reshape(NW, NCH_W, K)
    dstc = edge_index[1].reshape(NW, NCH_W, K)
    xp = jnp.concatenate([x, jnp.zeros((NPAD - N, D_IN), jnp.float32)], axis=0)
    zeros8 = jnp.zeros((RPS, 8), jnp.float32)
    ones8 = jnp.ones((K, 8), jnp.float32)
    zeros128 = jnp.zeros((RPS, D_H), jnp.float32)
    zeros16 = jnp.zeros((RPS, DOP), jnp.float32)
    W3p = jnp.concatenate([W3, jnp.zeros((D_H, DOP - D_OUT), jnp.float32)], axis=1)
    b3p = jnp.concatenate([b3, jnp.zeros((DOP - D_OUT,), jnp.float32)]).reshape(1, DOP)
    b1r = b1.reshape(1, D_H)
    b2r = b2.reshape(1, D_H)

    degp = _deg_kernel(dstc, ones8, zeros8)
    hs1, dinv = _tc1(xp, degp, W1)
    parts1 = _agg128(hs1, srcc, dstc, zeros128)
    hs2 = _tc_mid(parts1, hs1, dinv, b1r, W2, D_H)
    parts2 = _agg128(hs2, srcc, dstc, zeros128)
    hs3 = _tc_mid(parts2, hs2, dinv, b2r, W3p, DOP)
    parts3 = _agg16(hs3, srcc, dstc, zeros16)
    outp = _tc_fin(parts3, hs3, dinv, b3p)
    return outp[:N, :D_OUT]
